# baseline (speedup 1.0000x reference)
import jax
import jax.numpy as jnp
from jax import lax
from jax.experimental import pallas as pl
from jax.experimental.pallas import tpu as pltpu
from jax.experimental.pallas import tpu_sc as plsc

_NUM_WORDS = 1000000
_DIM = 64
_B = 4096
_CTX = 10
_NS = 21
_NC, _NSUB = 2, 16
_NW = _NC * _NSUB
_BW = _B // _NW
_YW = _BW * _NS
_CW = _BW * _CTX
_IDXN = _DIM * _NS + 16


def _dmm_body(ctxs_ref, docs_ref, y_ref, d_ref, w_ref, o_ref, out_ref,
              docs_v, ctx_v, y_v, docrows, ctxrows, idxbuf, colbuf, scores_v):
    wid = lax.axis_index("s") * _NC + lax.axis_index("c")

    pltpu.sync_copy(docs_ref.at[pl.ds(wid * _BW, _BW)], docs_v)
    pltpu.sync_copy(ctxs_ref.at[pl.ds(wid * _CW, _CW)], ctx_v)
    pltpu.sync_copy(y_ref.at[pl.ds(wid * _YW, _YW)], y_v.at[pl.ds(0, _YW)])

    pltpu.sync_copy(d_ref.at[docs_v], docrows)
    pltpu.sync_copy(w_ref.at[ctx_v], ctxrows)

    def x_body(b, carry):
        for q in range(_DIM // 16):
            acc0 = docrows[b, pl.ds(q * 16, 16)]
            acc1 = ctxrows[b * _CTX, pl.ds(q * 16, 16)]
            for c in range(1, _CTX, 2):
                acc0 = acc0 + ctxrows[b * _CTX + c, pl.ds(q * 16, 16)]
                if c + 1 < _CTX:
                    acc1 = acc1 + ctxrows[b * _CTX + c + 1, pl.ds(q * 16, 16)]
            docrows[b, pl.ds(q * 16, 16)] = acc0 + acc1
        return carry
    lax.fori_loop(0, _BW, x_body, 0)

    def z_body(i, carry):
        idxbuf[pl.ds(i * 16, 16)] = jnp.zeros((16,), jnp.int32)
        return carry
    lax.fori_loop(0, _IDXN // 16, z_body, 0)

    def s_body(b, carry):
        ya = y_v[pl.ds(b * _NS, 16)]
        yb = y_v[pl.ds(b * _NS + 16, 16)]
        for d in range(_DIM):
            off = jnp.int32(d * _NUM_WORDS)
            idxbuf[pl.ds(d * _NS, 16)] = ya + off
            idxbuf[pl.ds(d * _NS + 16, 16)] = yb + off
        pltpu.sync_copy(o_ref.at[idxbuf], colbuf)
        zero = jnp.zeros((16,), jnp.float32)
        a0 = [zero, zero]
        a1 = [zero, zero]
        xq = [docrows[b, pl.ds(q * 16, 16)] for q in range(_DIM // 16)]
        for d in range(_DIM):
            xs = xq[d // 16][d % 16]
            a0[d % 2] = a0[d % 2] + xs * colbuf[pl.ds(d * _NS, 16)]
            a1[d % 2] = a1[d % 2] + xs * colbuf[pl.ds(d * _NS + 16, 16)]
        scores_v[pl.ds(b * _NS, 16)] = a0[0] + a0[1]
        scores_v[pl.ds(b * _NS + 16, 16)] = a1[0] + a1[1]
        return carry
    lax.fori_loop(0, _BW, s_body, 0)

    pltpu.sync_copy(scores_v.at[pl.ds(0, _YW)], out_ref.at[pl.ds(wid * _YW, _YW)])


def kernel(ctxs, docs, y, D, W, O):
    ctxs_f = ctxs.reshape(-1).astype(jnp.int32)
    docs_i = docs.reshape(-1).astype(jnp.int32)
    y_f = y.reshape(-1).astype(jnp.int32)
    o_f = O.reshape(-1)
    run = pl.kernel(
        _dmm_body,
        out_type=jax.ShapeDtypeStruct((_B * _NS,), jnp.float32),
        mesh=plsc.VectorSubcoreMesh(
            core_axis_name="c", subcore_axis_name="s",
            num_cores=_NC, num_subcores=_NSUB),
        compiler_params=pltpu.CompilerParams(use_tc_tiling_on_sc=False),
        scratch_types=[
            pltpu.VMEM((_BW,), jnp.int32),
            pltpu.VMEM((_CW,), jnp.int32),
            pltpu.VMEM((_YW + 16,), jnp.int32),
            pltpu.VMEM((_BW, _DIM), jnp.float32),
            pltpu.VMEM((_CW, _DIM), jnp.float32),
            pltpu.VMEM((_IDXN,), jnp.int32),
            pltpu.VMEM((_IDXN,), jnp.float32),
            pltpu.VMEM((_YW + 16,), jnp.float32),
        ],
    )
    return run(ctxs_f, docs_i, y_f, D, W, o_f).reshape(_B, _NS)


# speedup vs baseline: 4.5073x; 4.5073x over previous
import jax
import jax.numpy as jnp
from jax import lax
from jax.experimental import pallas as pl
from jax.experimental.pallas import tpu as pltpu
from jax.experimental.pallas import tpu_sc as plsc

_DIM = 64
_B = 4096
_CTX = 10
_NS = 21
_NC, _NSUB = 2, 16
_NW = _NC * _NSUB
_BW = _B // _NW
_YW = _BW * _NS
_CW = _BW * _CTX
_CB = 16
_NCHUNK = _BW // _CB
_CROWS = _CB * _NS


def _dmm_body(ctxs_ref, docs_ref, y_ref, d_ref, w_ref, ot_ref, out_ref,
              docs_v, ctx_v, y_v, docrows, ctxrows, col0, col1, scores_v,
              sem0, sem1):
    wid = lax.axis_index("s") * _NC + lax.axis_index("c")

    pltpu.sync_copy(docs_ref.at[pl.ds(wid * _BW, _BW)], docs_v)
    pltpu.sync_copy(ctxs_ref.at[pl.ds(wid * _CW, _CW)], ctx_v)
    pltpu.sync_copy(y_ref.at[pl.ds(wid * _YW, _YW)], y_v.at[pl.ds(0, _YW)])

    pltpu.sync_copy(d_ref.at[docs_v], docrows)
    half = _CW // 2
    cols = [col0, col1]
    sems = [sem0, sem1]

    descs = [None, None]
    descs[0] = pltpu.async_copy(ot_ref.at[y_v.at[pl.ds(0, _CROWS)]], col0, sem0)

    for h in range(2):
        pltpu.sync_copy(w_ref.at[ctx_v.at[pl.ds(h * half, half)]],
                        ctxrows.at[pl.ds(0, half)])

        def x_body(b, carry, h=h):
            gb = h * (_BW // 2) + b
            for q in range(_DIM // 16):
                acc0 = docrows[gb, pl.ds(q * 16, 16)]
                acc1 = ctxrows[b * _CTX, pl.ds(q * 16, 16)]
                for c in range(1, _CTX, 2):
                    acc0 = acc0 + ctxrows[b * _CTX + c, pl.ds(q * 16, 16)]
                    if c + 1 < _CTX:
                        acc1 = acc1 + ctxrows[b * _CTX + c + 1, pl.ds(q * 16, 16)]
                docrows[gb, pl.ds(q * 16, 16)] = acc0 + acc1
            return carry
        lax.fori_loop(0, _BW // 2, x_body, 0)

    for c in range(_NCHUNK):
        if c + 1 < _NCHUNK:
            nsel = (c + 1) % 2
            descs[nsel] = pltpu.async_copy(
                ot_ref.at[y_v.at[pl.ds((c + 1) * _CROWS, _CROWS)]],
                cols[nsel], sems[nsel])
        sel = c % 2
        descs[sel].wait()
        colbuf = cols[sel]

        def dot_body(g, carry, c=c, colbuf=colbuf):
            lanes = lax.iota(jnp.int32, 16)
            rvec = lanes + g * 16
            bvec = (rvec + c * _CROWS) // _NS
            acc0 = jnp.zeros((16,), jnp.float32)
            acc1 = jnp.zeros((16,), jnp.float32)
            for d in range(_DIM):
                dsplat = jnp.full((16,), d, jnp.int32)
                ov = plsc.load_gather(colbuf, [rvec, dsplat])
                xv = plsc.load_gather(docrows, [bvec, dsplat])
                if d % 2 == 0:
                    acc0 = acc0 + xv * ov
                else:
                    acc1 = acc1 + xv * ov
            scores_v[pl.ds(c * _CROWS + g * 16, 16)] = acc0 + acc1
            return carry
        lax.fori_loop(0, _CROWS // 16, dot_body, 0)

    pltpu.sync_copy(scores_v.at[pl.ds(0, _YW)], out_ref.at[pl.ds(wid * _YW, _YW)])


def kernel(ctxs, docs, y, D, W, O):
    ctxs_f = ctxs.reshape(-1).astype(jnp.int32)
    docs_i = docs.reshape(-1).astype(jnp.int32)
    y_f = y.reshape(-1).astype(jnp.int32)
    ot = O.T
    run = pl.kernel(
        _dmm_body,
        out_type=jax.ShapeDtypeStruct((_B * _NS,), jnp.float32),
        mesh=plsc.VectorSubcoreMesh(
            core_axis_name="c", subcore_axis_name="s",
            num_cores=_NC, num_subcores=_NSUB),
        compiler_params=pltpu.CompilerParams(
            use_tc_tiling_on_sc=False, needs_layout_passes=False),
        scratch_types=[
            pltpu.VMEM((_BW,), jnp.int32),
            pltpu.VMEM((_CW,), jnp.int32),
            pltpu.VMEM((_YW + 16,), jnp.int32),
            pltpu.VMEM((_BW, _DIM), jnp.float32),
            pltpu.VMEM((_CW // 2, _DIM), jnp.float32),
            pltpu.VMEM((_CROWS, _DIM), jnp.float32),
            pltpu.VMEM((_CROWS, _DIM), jnp.float32),
            pltpu.VMEM((_YW + 16,), jnp.float32),
            pltpu.SemaphoreType.DMA,
            pltpu.SemaphoreType.DMA,
        ],
    )
    return run(ctxs_f, docs_i, y_f, D, W, ot).reshape(_B, _NS)
